# Initial kernel scaffold; baseline (speedup 1.0000x reference)
import jax
import jax.numpy as jnp
from jax.experimental import pallas as pl


def kernel(x, edge_index, batch, atom_emb, W1, b1, W2, b2, W3, b3, Wc1, bc1, Wc2, bc2):
    raise NotImplementedError("write your pallas kernel here")


# speedup vs baseline: 2.1296x; 2.1296x over previous
import functools

import jax
import jax.numpy as jnp
from jax import lax
from jax.experimental import pallas as pl
from jax.experimental.pallas import tpu as pltpu
from jax.experimental.pallas import tpu_sc as plsc

N = 50000
E = 800000
H = 128
G = 128
F = 9
VOC = 128

NC = 2
NS = 16
NPART = 7
PART = 7168
NPACC = 7176
NROWS = NPART * PART
TRASH = 50040
IB = 128
KB = 4
STEP_E = IB * KB
T_TILE = 26624
NSTEP = T_TILE // STEP_E
EP = T_TILE * NC * NS
ZROWS = PART // NS

BN = 1000
NBLK = N // BN

_HIGH = jax.lax.Precision.HIGHEST


def _dot(a, b, dims=None):
    if dims is None:
        return lax.dot_general(a, b, (((1,), (0,)), ((), ())),
                               precision=_HIGH, preferred_element_type=jnp.float32)
    return lax.dot_general(a, b, (dims, ((), ())),
                           precision=_HIGH, preferred_element_type=jnp.float32)


def _enc_body(x_ref, emb_ref, o_ref):
    oh_iota = lax.broadcasted_iota(jnp.int32, (BN, VOC), 1)
    acc = jnp.zeros((BN, H), jnp.float32)
    for i in range(F):
        oh = (x_ref[:, i:i + 1] == oh_iota).astype(jnp.float32)
        acc = acc + _dot(oh, emb_ref[i])
    o_ref[:, :] = acc


def _atom_encode(x, atom_emb):
    return pl.pallas_call(
        _enc_body,
        grid=(NBLK,),
        in_specs=[
            pl.BlockSpec((BN, F), lambda i: (i, 0)),
            pl.BlockSpec((F, VOC, H), lambda i: (0, 0, 0)),
        ],
        out_specs=pl.BlockSpec((BN, H), lambda i: (i, 0)),
        out_shape=jax.ShapeDtypeStruct((N, H), jnp.float32),
    )(x, atom_emb)


def _scale1_body(h_ref, w_ref, deg_ref, g_ref):
    dinv = lax.rsqrt(deg_ref[:, :1])
    g_ref[:, :] = dinv * _dot(h_ref[:, :], w_ref[:, :])


def _scale1(h0, W1, deg):
    return pl.pallas_call(
        _scale1_body,
        grid=(NBLK,),
        in_specs=[
            pl.BlockSpec((BN, H), lambda i: (i, 0)),
            pl.BlockSpec((H, H), lambda i: (0, 0)),
            pl.BlockSpec((BN, H), lambda i: (i, 0)),
        ],
        out_specs=pl.BlockSpec((BN, H), lambda i: (i, 0)),
        out_shape=jax.ShapeDtypeStruct((N, H), jnp.float32),
    )(h0, W1, deg)


def _mid_body(a0_ref, a1_ref, deg_ref, b_ref, w_ref, xs_ref, xsum_ref, g_ref):
    dinv = lax.rsqrt(deg_ref[:, :1])
    xl = jnp.maximum((a0_ref[:, :] + a1_ref[:, :]) * dinv + b_ref[:, :], 0.0)
    xsum_ref[:, :] = xs_ref[:, :] + xl if xs_ref is not None else xl
    g_ref[:, :] = dinv * _dot(xl, w_ref[:, :])


def _mid_first_body(a0_ref, a1_ref, deg_ref, b_ref, w_ref, xsum_ref, g_ref):
    _mid_body(a0_ref, a1_ref, deg_ref, b_ref, w_ref, None, xsum_ref, g_ref)


def _merge_mid(A0, A1, deg, b, W_next, xsum_in):
    row = lambda i: (i, 0)
    fix = lambda i: (0, 0)
    in_specs = [pl.BlockSpec((BN, H), row), pl.BlockSpec((BN, H), row),
                pl.BlockSpec((BN, H), row), pl.BlockSpec((1, H), fix),
                pl.BlockSpec((H, H), fix)]
    args = [A0, A1, deg, b, W_next]
    body = _mid_first_body
    if xsum_in is not None:
        in_specs.append(pl.BlockSpec((BN, H), row))
        args.append(xsum_in)

        def body(a0, a1, dg, bb, ww, xs, xo, go):
            _mid_body(a0, a1, dg, bb, ww, xs, xo, go)
    return pl.pallas_call(
        body,
        grid=(NBLK,),
        in_specs=in_specs,
        out_specs=[pl.BlockSpec((BN, H), row), pl.BlockSpec((BN, H), row)],
        out_shape=[jax.ShapeDtypeStruct((N, H), jnp.float32),
                   jax.ShapeDtypeStruct((N, H), jnp.float32)],
    )(*args)


def _last_body(a0_ref, a1_ref, deg_ref, b_ref, xs_ref, xsum_ref):
    dinv = lax.rsqrt(deg_ref[:, :1])
    xl = jnp.maximum((a0_ref[:, :] + a1_ref[:, :]) * dinv + b_ref[:, :], 0.0)
    xsum_ref[:, :] = xs_ref[:, :] + xl


def _merge_last(A0, A1, deg, b, xsum_in):
    row = lambda i: (i, 0)
    return pl.pallas_call(
        _last_body,
        grid=(NBLK,),
        in_specs=[pl.BlockSpec((BN, H), row), pl.BlockSpec((BN, H), row),
                  pl.BlockSpec((BN, H), row), pl.BlockSpec((1, H), lambda i: (0, 0)),
                  pl.BlockSpec((BN, H), row)],
        out_specs=pl.BlockSpec((BN, H), row),
        out_shape=jax.ShapeDtypeStruct((N, H), jnp.float32),
    )(A0, A1, deg, b, xsum_in)


def _pool_body(batch_ref, x_ref, pooled_ref, cnt_ref):
    i = pl.program_id(0)

    @pl.when(i == 0)
    def _init():
        pooled_ref[:, :] = jnp.zeros((G, H), jnp.float32)
        cnt_ref[:, :] = jnp.zeros((G, H), jnp.float32)

    oh = (batch_ref[:, :] == lax.broadcasted_iota(jnp.int32, (BN, G), 1))
    oh = oh.astype(jnp.float32)
    pooled_ref[:, :] += _dot(oh, x_ref[:, :], dims=((0,), (0,)))
    cnt_ref[:, :] += _dot(oh, jnp.ones((BN, H), jnp.float32), dims=((0,), (0,)))


def _pool(batch2d, xsum):
    return pl.pallas_call(
        _pool_body,
        grid=(NBLK,),
        in_specs=[pl.BlockSpec((BN, 1), lambda i: (i, 0)),
                  pl.BlockSpec((BN, H), lambda i: (i, 0))],
        out_specs=[pl.BlockSpec((G, H), lambda i: (0, 0)),
                   pl.BlockSpec((G, H), lambda i: (0, 0))],
        out_shape=[jax.ShapeDtypeStruct((G, H), jnp.float32),
                   jax.ShapeDtypeStruct((G, H), jnp.float32)],
    )(batch2d, xsum)


def _head_body(ps_ref, cnt_ref, w1_ref, b1_ref, w2_ref, b2_ref, o_ref):
    pooled = ps_ref[:, :] / jnp.maximum(cnt_ref[:, :], 1.0)
    hid = jnp.maximum(_dot(pooled, w1_ref[:, :]) + b1_ref[:, :], 0.0)
    o_ref[:, :] = _dot(hid, w2_ref[:, :]) + b2_ref[:, :]


def _head(pooled_sum, cnt, Wc1, bc1, Wc2, bc2):
    return pl.pallas_call(
        _head_body,
        out_shape=jax.ShapeDtypeStruct((G, 10), jnp.float32),
    )(pooled_sum, cnt, Wc1, bc1, Wc2, bc2)


_MESH = dict(core_axis_name="c", subcore_axis_name="s", num_cores=NC,
             num_subcores=NS)


def _clamp_dst(dst_i, loc_i, p):
    base = p * PART
    for j in range(KB):
        for cc in range(IB // 16):
            v = dst_i[j, pl.ds(cc * 16, 16)]
            t = v - base
            ok = (t >= 0) & (t < PART)
            loc_i[j, pl.ds(cc * 16, 16)] = jnp.where(ok, t, PART)


def _deg_kernel(dsts2d, ones_h, zeros_h):
    @functools.partial(
        pl.kernel,
        mesh=plsc.VectorSubcoreMesh(**_MESH),
        out_type=[jax.ShapeDtypeStruct((NROWS, H), jnp.float32),
                  jax.ShapeDtypeStruct((NROWS, H), jnp.float32)],
        scratch_types=[
            pltpu.VMEM((KB, IB), jnp.int32),
            pltpu.VMEM((KB, IB), jnp.int32),
            pltpu.VMEM((IB, H), jnp.float32),
            pltpu.VMEM_SHARED((NPACC, H), jnp.float32),
        ],
    )
    def body(dsts_ref, ones_ref, zeros_ref, out0, out1, dst_i, loc_i, ones_v,
             accum):
        cid = lax.axis_index("c")
        sid = lax.axis_index("s")
        tile = cid * NS + sid
        r0 = sid * ZROWS

        pltpu.sync_copy(ones_ref, ones_v)
        for p in range(NPART):
            pltpu.sync_copy(zeros_ref, accum.at[pl.ds(r0, ZROWS)])
            plsc.subcore_barrier()

            def step(k, carry):
                row0 = tile * (T_TILE // IB) + k * KB
                pltpu.sync_copy(dsts_ref.at[pl.ds(row0, KB)], dst_i)
                _clamp_dst(dst_i, loc_i, p)
                for j in range(KB):
                    pltpu.sync_copy(ones_v, accum.at[loc_i.at[j]], add=True)
                return carry

            lax.fori_loop(0, NSTEP, step, 0)
            plsc.subcore_barrier()

            @pl.when(cid == 0)
            def _d0():
                pltpu.sync_copy(accum.at[pl.ds(r0, ZROWS)],
                                out0.at[pl.ds(p * PART + r0, ZROWS)])

            @pl.when(cid == 1)
            def _d1():
                pltpu.sync_copy(accum.at[pl.ds(r0, ZROWS)],
                                out1.at[pl.ds(p * PART + r0, ZROWS)])
            plsc.subcore_barrier()

    return body(dsts2d, ones_h, zeros_h)


def _edge_kernel(g, srcs2d, dsts2d, zeros_h):
    @functools.partial(
        pl.kernel,
        mesh=plsc.VectorSubcoreMesh(**_MESH),
        out_type=[jax.ShapeDtypeStruct((NROWS, H), jnp.float32),
                  jax.ShapeDtypeStruct((NROWS, H), jnp.float32)],
        scratch_types=[
            pltpu.VMEM((KB, IB), jnp.int32),
            pltpu.VMEM((KB, IB), jnp.int32),
            pltpu.VMEM((KB, IB), jnp.int32),
            pltpu.VMEM((KB, IB, H), jnp.float32),
            pltpu.VMEM_SHARED((NPACC, H), jnp.float32),
            pltpu.SemaphoreType.DMA,
        ],
    )
    def body(g_ref, srcs_ref, dsts_ref, zeros_ref, out0, out1,
             src_i, dst_i, loc_i, rows_v, accum, sem):
        cid = lax.axis_index("c")
        sid = lax.axis_index("s")
        tile = cid * NS + sid
        r0 = sid * ZROWS

        for p in range(NPART):
            pltpu.sync_copy(zeros_ref, accum.at[pl.ds(r0, ZROWS)])
            plsc.subcore_barrier()

            def step(k, carry):
                row0 = tile * (T_TILE // IB) + k * KB
                pltpu.sync_copy(srcs_ref.at[pl.ds(row0, KB)], src_i)
                pltpu.sync_copy(dsts_ref.at[pl.ds(row0, KB)], dst_i)
                _clamp_dst(dst_i, loc_i, p)
                descs = [
                    pltpu.async_copy(g_ref.at[src_i.at[j]], rows_v.at[j], sem)
                    for j in range(KB)
                ]
                for d in descs:
                    d.wait()
                for j in range(KB):
                    pltpu.sync_copy(rows_v.at[j], accum.at[loc_i.at[j]],
                                    add=True)
                return carry

            lax.fori_loop(0, NSTEP, step, 0)
            plsc.subcore_barrier()

            @pl.when(cid == 0)
            def _d0():
                pltpu.sync_copy(accum.at[pl.ds(r0, ZROWS)],
                                out0.at[pl.ds(p * PART + r0, ZROWS)])

            @pl.when(cid == 1)
            def _d1():
                pltpu.sync_copy(accum.at[pl.ds(r0, ZROWS)],
                                out1.at[pl.ds(p * PART + r0, ZROWS)])
            plsc.subcore_barrier()

    return body(g, srcs2d, dsts2d, zeros_h)


def kernel(x, edge_index, batch, atom_emb, W1, b1, W2, b2, W3, b3,
           Wc1, bc1, Wc2, bc2):
    pad = EP - (E + N)
    loop = jnp.arange(N, dtype=jnp.int32)
    srcs = jnp.concatenate([edge_index[0], loop,
                            jnp.zeros((pad,), jnp.int32)]).reshape(EP // IB, IB)
    dsts = jnp.concatenate([edge_index[1], loop,
                            jnp.full((pad,), TRASH, jnp.int32)]).reshape(EP // IB, IB)
    zeros_h = jnp.zeros((ZROWS, H), jnp.float32)
    ones_h = jnp.ones((IB, H), jnp.float32)

    h0 = _atom_encode(x, atom_emb)
    d0, d1 = _deg_kernel(dsts, ones_h, zeros_h)
    deg = d0 + d1

    g1 = _scale1(h0, W1, deg)
    A0, A1 = _edge_kernel(g1, srcs, dsts, zeros_h)
    xsum1, g2 = _merge_mid(A0, A1, deg, b1.reshape(1, H), W2, None)
    B0, B1 = _edge_kernel(g2, srcs, dsts, zeros_h)
    xsum2, g3 = _merge_mid(B0, B1, deg, b2.reshape(1, H), W3, xsum1)
    C0, C1 = _edge_kernel(g3, srcs, dsts, zeros_h)
    xsum3 = _merge_last(C0, C1, deg, b3.reshape(1, H), xsum2)

    pooled_sum, cnt = _pool(batch.reshape(N, 1), xsum3)
    out = _head(pooled_sum, cnt, Wc1, bc1.reshape(1, H // 2),
                Wc2, bc2.reshape(1, 10))
    return (out, 0)


# speedup vs baseline: 2.2398x; 1.0517x over previous
import functools

import jax
import jax.numpy as jnp
from jax import lax
from jax.experimental import pallas as pl
from jax.experimental.pallas import tpu as pltpu
from jax.experimental.pallas import tpu_sc as plsc

N = 50000
E = 800000
H = 128
G = 128
F = 9
VOC = 128

NC = 2
NS = 16
NPART = 7
PART = 7168
NPACC = 7176
NROWS = NPART * PART
TRASH = 50040
IB = 128
KB = 2
STEP_E = IB * KB
T_TILE = 26624
NSTEP = T_TILE // STEP_E
EP = T_TILE * NC * NS
ZROWS = PART // NS

BN = 1000
NBLK = N // BN

_HIGH = jax.lax.Precision.HIGHEST


def _dot(a, b, dims=None):
    if dims is None:
        return lax.dot_general(a, b, (((1,), (0,)), ((), ())),
                               precision=_HIGH, preferred_element_type=jnp.float32)
    return lax.dot_general(a, b, (dims, ((), ())),
                           precision=_HIGH, preferred_element_type=jnp.float32)


def _enc_body(x_ref, emb_ref, o_ref):
    oh_iota = lax.broadcasted_iota(jnp.int32, (BN, VOC), 1)
    acc = jnp.zeros((BN, H), jnp.float32)
    for i in range(F):
        oh = (x_ref[:, i:i + 1] == oh_iota).astype(jnp.float32)
        acc = acc + _dot(oh, emb_ref[i])
    o_ref[:, :] = acc


def _atom_encode(x, atom_emb):
    return pl.pallas_call(
        _enc_body,
        grid=(NBLK,),
        in_specs=[
            pl.BlockSpec((BN, F), lambda i: (i, 0)),
            pl.BlockSpec((F, VOC, H), lambda i: (0, 0, 0)),
        ],
        out_specs=pl.BlockSpec((BN, H), lambda i: (i, 0)),
        out_shape=jax.ShapeDtypeStruct((N, H), jnp.float32),
    )(x, atom_emb)


def _scale1_body(h_ref, w_ref, deg_ref, g_ref):
    dinv = lax.rsqrt(deg_ref[:, :1])
    g_ref[:, :] = dinv * _dot(h_ref[:, :], w_ref[:, :])


def _scale1(h0, W1, deg):
    return pl.pallas_call(
        _scale1_body,
        grid=(NBLK,),
        in_specs=[
            pl.BlockSpec((BN, H), lambda i: (i, 0)),
            pl.BlockSpec((H, H), lambda i: (0, 0)),
            pl.BlockSpec((BN, H), lambda i: (i, 0)),
        ],
        out_specs=pl.BlockSpec((BN, H), lambda i: (i, 0)),
        out_shape=jax.ShapeDtypeStruct((N, H), jnp.float32),
    )(h0, W1, deg)


def _mid_body(a0_ref, a1_ref, deg_ref, b_ref, w_ref, xs_ref, xsum_ref, g_ref):
    dinv = lax.rsqrt(deg_ref[:, :1])
    xl = jnp.maximum((a0_ref[:, :] + a1_ref[:, :]) * dinv + b_ref[:, :], 0.0)
    xsum_ref[:, :] = xs_ref[:, :] + xl if xs_ref is not None else xl
    g_ref[:, :] = dinv * _dot(xl, w_ref[:, :])


def _mid_first_body(a0_ref, a1_ref, deg_ref, b_ref, w_ref, xsum_ref, g_ref):
    _mid_body(a0_ref, a1_ref, deg_ref, b_ref, w_ref, None, xsum_ref, g_ref)


def _merge_mid(A0, A1, deg, b, W_next, xsum_in):
    row = lambda i: (i, 0)
    fix = lambda i: (0, 0)
    in_specs = [pl.BlockSpec((BN, H), row), pl.BlockSpec((BN, H), row),
                pl.BlockSpec((BN, H), row), pl.BlockSpec((1, H), fix),
                pl.BlockSpec((H, H), fix)]
    args = [A0, A1, deg, b, W_next]
    body = _mid_first_body
    if xsum_in is not None:
        in_specs.append(pl.BlockSpec((BN, H), row))
        args.append(xsum_in)

        def body(a0, a1, dg, bb, ww, xs, xo, go):
            _mid_body(a0, a1, dg, bb, ww, xs, xo, go)
    return pl.pallas_call(
        body,
        grid=(NBLK,),
        in_specs=in_specs,
        out_specs=[pl.BlockSpec((BN, H), row), pl.BlockSpec((BN, H), row)],
        out_shape=[jax.ShapeDtypeStruct((N, H), jnp.float32),
                   jax.ShapeDtypeStruct((N, H), jnp.float32)],
    )(*args)


def _last_body(a0_ref, a1_ref, deg_ref, b_ref, xs_ref, xsum_ref):
    dinv = lax.rsqrt(deg_ref[:, :1])
    xl = jnp.maximum((a0_ref[:, :] + a1_ref[:, :]) * dinv + b_ref[:, :], 0.0)
    xsum_ref[:, :] = xs_ref[:, :] + xl


def _merge_last(A0, A1, deg, b, xsum_in):
    row = lambda i: (i, 0)
    return pl.pallas_call(
        _last_body,
        grid=(NBLK,),
        in_specs=[pl.BlockSpec((BN, H), row), pl.BlockSpec((BN, H), row),
                  pl.BlockSpec((BN, H), row), pl.BlockSpec((1, H), lambda i: (0, 0)),
                  pl.BlockSpec((BN, H), row)],
        out_specs=pl.BlockSpec((BN, H), row),
        out_shape=jax.ShapeDtypeStruct((N, H), jnp.float32),
    )(A0, A1, deg, b, xsum_in)


def _pool_body(batch_ref, x_ref, pooled_ref, cnt_ref):
    i = pl.program_id(0)

    @pl.when(i == 0)
    def _init():
        pooled_ref[:, :] = jnp.zeros((G, H), jnp.float32)
        cnt_ref[:, :] = jnp.zeros((G, H), jnp.float32)

    oh = (batch_ref[:, :] == lax.broadcasted_iota(jnp.int32, (BN, G), 1))
    oh = oh.astype(jnp.float32)
    pooled_ref[:, :] += _dot(oh, x_ref[:, :], dims=((0,), (0,)))
    cnt_ref[:, :] += _dot(oh, jnp.ones((BN, H), jnp.float32), dims=((0,), (0,)))


def _pool(batch2d, xsum):
    return pl.pallas_call(
        _pool_body,
        grid=(NBLK,),
        in_specs=[pl.BlockSpec((BN, 1), lambda i: (i, 0)),
                  pl.BlockSpec((BN, H), lambda i: (i, 0))],
        out_specs=[pl.BlockSpec((G, H), lambda i: (0, 0)),
                   pl.BlockSpec((G, H), lambda i: (0, 0))],
        out_shape=[jax.ShapeDtypeStruct((G, H), jnp.float32),
                   jax.ShapeDtypeStruct((G, H), jnp.float32)],
    )(batch2d, xsum)


def _head_body(ps_ref, cnt_ref, w1_ref, b1_ref, w2_ref, b2_ref, o_ref):
    pooled = ps_ref[:, :] / jnp.maximum(cnt_ref[:, :], 1.0)
    hid = jnp.maximum(_dot(pooled, w1_ref[:, :]) + b1_ref[:, :], 0.0)
    o_ref[:, :] = _dot(hid, w2_ref[:, :]) + b2_ref[:, :]


def _head(pooled_sum, cnt, Wc1, bc1, Wc2, bc2):
    return pl.pallas_call(
        _head_body,
        out_shape=jax.ShapeDtypeStruct((G, 10), jnp.float32),
    )(pooled_sum, cnt, Wc1, bc1, Wc2, bc2)


_MESH = dict(core_axis_name="c", subcore_axis_name="s", num_cores=NC,
             num_subcores=NS)


def _clamp_dst(dst_i, loc_i, nb, p):
    base = p * PART
    for j in range(KB):
        for cc in range(IB // 16):
            v = dst_i[nb, j, pl.ds(cc * 16, 16)]
            t = v - base
            ok = (t >= 0) & (t < PART)
            loc_i[nb, j, pl.ds(cc * 16, 16)] = jnp.where(ok, t, PART)


def _deg_kernel(dsts2d, ones_h, zeros_h):
    @functools.partial(
        pl.kernel,
        mesh=plsc.VectorSubcoreMesh(**_MESH),
        out_type=[jax.ShapeDtypeStruct((NROWS, H), jnp.float32),
                  jax.ShapeDtypeStruct((NROWS, H), jnp.float32)],
        scratch_types=[
            pltpu.VMEM((2, KB, IB), jnp.int32),
            pltpu.VMEM((2, KB, IB), jnp.int32),
            pltpu.VMEM((IB, H), jnp.float32),
            pltpu.VMEM_SHARED((NPACC, H), jnp.float32),
            pltpu.SemaphoreType.DMA,
        ],
    )
    def body(dsts_ref, ones_ref, zeros_ref, out0, out1, dst_i, loc_i, ones_v,
             accum, sem_s):
        cid = lax.axis_index("c")
        sid = lax.axis_index("s")
        tile = cid * NS + sid
        r0 = sid * ZROWS

        def drain_one():
            pltpu.make_async_copy(zeros_ref.at[pl.ds(0, IB)], ones_v,
                                  sem_s).wait()

        pltpu.sync_copy(ones_ref, ones_v)
        for p in range(NPART):
            pltpu.sync_copy(zeros_ref.at[pl.ds(0, ZROWS)],
                            accum.at[pl.ds(r0, ZROWS)])
            plsc.subcore_barrier()

            pltpu.sync_copy(dsts_ref.at[pl.ds(tile * (T_TILE // IB), KB)],
                            dst_i.at[0])
            _clamp_dst(dst_i, loc_i, 0, p)

            def halfstep(k, b):
                nb = 1 - b

                @pl.when(k > 0)
                def _drain_prev():
                    for _ in range(KB):
                        drain_one()

                @pl.when(k < NSTEP - 1)
                def _load_next():
                    row0 = tile * (T_TILE // IB) + (k + 1) * KB
                    pltpu.sync_copy(dsts_ref.at[pl.ds(row0, KB)], dst_i.at[nb])
                    _clamp_dst(dst_i, loc_i, nb, p)

                for j in range(KB):
                    pltpu.async_copy(ones_v, accum.at[loc_i.at[b, j]], sem_s,
                                     add=True)

            def step(kk, carry):
                halfstep(2 * kk, 0)
                halfstep(2 * kk + 1, 1)
                return carry

            lax.fori_loop(0, NSTEP // 2, step, 0)
            for _ in range(KB):
                drain_one()
            plsc.subcore_barrier()

            @pl.when(cid == 0)
            def _d0():
                pltpu.sync_copy(accum.at[pl.ds(r0, ZROWS)],
                                out0.at[pl.ds(p * PART + r0, ZROWS)])

            @pl.when(cid == 1)
            def _d1():
                pltpu.sync_copy(accum.at[pl.ds(r0, ZROWS)],
                                out1.at[pl.ds(p * PART + r0, ZROWS)])
            plsc.subcore_barrier()

    return body(dsts2d, ones_h, zeros_h)


def _edge_kernel(g, srcs2d, dsts2d, zeros_h):
    @functools.partial(
        pl.kernel,
        mesh=plsc.VectorSubcoreMesh(**_MESH),
        out_type=[jax.ShapeDtypeStruct((NROWS, H), jnp.float32),
                  jax.ShapeDtypeStruct((NROWS, H), jnp.float32)],
        scratch_types=[
            pltpu.VMEM((2, KB, IB), jnp.int32),
            pltpu.VMEM((2, KB, IB), jnp.int32),
            pltpu.VMEM((2, KB, IB), jnp.int32),
            pltpu.VMEM((2, KB, IB, H), jnp.float32),
            pltpu.VMEM_SHARED((NPACC, H), jnp.float32),
            pltpu.SemaphoreType.DMA,
            pltpu.SemaphoreType.DMA,
        ],
    )
    def body(g_ref, srcs_ref, dsts_ref, zeros_ref, out0, out1,
             src_i, dst_i, loc_i, rows_v, accum, sem_g, sem_s):
        cid = lax.axis_index("c")
        sid = lax.axis_index("s")
        tile = cid * NS + sid
        r0 = sid * ZROWS

        def drain(sem, n):
            for _ in range(n):
                pltpu.make_async_copy(zeros_ref.at[pl.ds(0, IB)],
                                      rows_v.at[0, 0], sem).wait()

        for p in range(NPART):
            pltpu.sync_copy(zeros_ref.at[pl.ds(0, ZROWS)],
                            accum.at[pl.ds(r0, ZROWS)])
            plsc.subcore_barrier()

            row00 = tile * (T_TILE // IB)
            pltpu.sync_copy(srcs_ref.at[pl.ds(row00, KB)], src_i.at[0])
            pltpu.sync_copy(dsts_ref.at[pl.ds(row00, KB)], dst_i.at[0])
            _clamp_dst(dst_i, loc_i, 0, p)
            for j in range(KB):
                pltpu.async_copy(g_ref.at[src_i.at[0, j]], rows_v.at[0, j],
                                 sem_g)

            def halfstep(k, b):
                nb = 1 - b

                @pl.when(k > 0)
                def _drain_prev_scatters():
                    drain(sem_s, KB)

                @pl.when(k < NSTEP - 1)
                def _load_next_idx():
                    row0 = tile * (T_TILE // IB) + (k + 1) * KB
                    pltpu.sync_copy(srcs_ref.at[pl.ds(row0, KB)],
                                    src_i.at[nb])
                    pltpu.sync_copy(dsts_ref.at[pl.ds(row0, KB)],
                                    dst_i.at[nb])
                    _clamp_dst(dst_i, loc_i, nb, p)

                drain(sem_g, KB)
                for j in range(KB):
                    pltpu.async_copy(rows_v.at[b, j],
                                     accum.at[loc_i.at[b, j]], sem_s,
                                     add=True)

                @pl.when(k < NSTEP - 1)
                def _fire_next_gathers():
                    for j in range(KB):
                        pltpu.async_copy(g_ref.at[src_i.at[nb, j]],
                                         rows_v.at[nb, j], sem_g)

            def step(kk, carry):
                halfstep(2 * kk, 0)
                halfstep(2 * kk + 1, 1)
                return carry

            lax.fori_loop(0, NSTEP // 2, step, 0)
            drain(sem_s, KB)
            plsc.subcore_barrier()

            @pl.when(cid == 0)
            def _d0():
                pltpu.sync_copy(accum.at[pl.ds(r0, ZROWS)],
                                out0.at[pl.ds(p * PART + r0, ZROWS)])

            @pl.when(cid == 1)
            def _d1():
                pltpu.sync_copy(accum.at[pl.ds(r0, ZROWS)],
                                out1.at[pl.ds(p * PART + r0, ZROWS)])
            plsc.subcore_barrier()

    return body(g, srcs2d, dsts2d, zeros_h)


def kernel(x, edge_index, batch, atom_emb, W1, b1, W2, b2, W3, b3,
           Wc1, bc1, Wc2, bc2):
    pad = EP - (E + N)
    loop = jnp.arange(N, dtype=jnp.int32)
    srcs = jnp.concatenate([edge_index[0], loop,
                            jnp.zeros((pad,), jnp.int32)]).reshape(EP // IB, IB)
    dsts = jnp.concatenate([edge_index[1], loop,
                            jnp.full((pad,), TRASH, jnp.int32)]).reshape(EP // IB, IB)
    zeros_h = jnp.zeros((ZROWS, H), jnp.float32)
    ones_h = jnp.ones((IB, H), jnp.float32)

    h0 = _atom_encode(x, atom_emb)
    d0, d1 = _deg_kernel(dsts, ones_h, zeros_h)
    deg = d0 + d1

    g1 = _scale1(h0, W1, deg)
    A0, A1 = _edge_kernel(g1, srcs, dsts, zeros_h)
    xsum1, g2 = _merge_mid(A0, A1, deg, b1.reshape(1, H), W2, None)
    B0, B1 = _edge_kernel(g2, srcs, dsts, zeros_h)
    xsum2, g3 = _merge_mid(B0, B1, deg, b2.reshape(1, H), W3, xsum1)
    C0, C1 = _edge_kernel(g3, srcs, dsts, zeros_h)
    xsum3 = _merge_last(C0, C1, deg, b3.reshape(1, H), xsum2)

    pooled_sum, cnt = _pool(batch.reshape(N, 1), xsum3)
    out = _head(pooled_sum, cnt, Wc1, bc1.reshape(1, H // 2),
                Wc2, bc2.reshape(1, 10))
    return (out, 0)
